# baseline (speedup 1.0000x reference)
import functools

import jax
import jax.numpy as jnp
from jax import lax
from jax.experimental import pallas as pl
from jax.experimental.pallas import tpu as pltpu
from jax.experimental.pallas import tpu_sc as plsc

_NC = 2
_NS = 16
_NW = _NC * _NS
_L = 16
_K = 8
_W = _L * _K


@functools.lru_cache(maxsize=None)
def _make_sc_pass(n_rows: int, groups: int):
    chunk = groups * _L
    total_chunks = n_rows // chunk
    main_chunks = (total_chunks // _NW) * _NW
    per_w = main_chunks // _NW
    npairs = per_w // 2
    tail = total_chunks - main_chunks
    assert total_chunks * chunk == n_rows
    assert npairs * 2 == per_w
    assert tail < _NW

    mesh = plsc.VectorSubcoreMesh(core_axis_name="c", subcore_axis_name="s")

    @functools.partial(
        pl.kernel,
        mesh=mesh,
        out_type=jax.ShapeDtypeStruct((_NW, _L), jnp.float32),
        scratch_types=[
            pltpu.VMEM((groups, _W), jnp.float32),
            pltpu.VMEM((groups, _W), jnp.float32),
            pltpu.VMEM((chunk,), jnp.int32),
            pltpu.VMEM((chunk,), jnp.int32),
            pltpu.VMEM((_L * _K,), jnp.float32),
            pltpu.VMEM((_L * _K,), jnp.float32),
            pltpu.VMEM((_L,), jnp.float32),
            pltpu.SemaphoreType.DMA,
            pltpu.SemaphoreType.DMA,
        ],
        compiler_params=pltpu.CompilerParams(needs_layout_passes=False),
    )
    def sc_pass(logits_hbm, a_hbm, out_hbm,
                lbuf0, lbuf1, abuf0, abuf1,
                acc_tbl, cnt_tbl, obuf, sem0, sem1):
        lbufs = (lbuf0, lbuf1)
        abufs = (abuf0, abuf1)
        wid = lax.axis_index("s") * _NC + lax.axis_index("c")

        iota = lax.iota(jnp.int32, _L)
        i8 = iota * _K
        zeros = jnp.zeros((_L,), jnp.float32)
        ones = jnp.ones((_L,), jnp.float32)
        cols = [i8 + j for j in range(_K)]

        for k in range(_K):
            acc_tbl[pl.ds(k * _L, _L)] = zeros
            cnt_tbl[pl.ds(k * _L, _L)] = zeros

        def start_copy(cid, b, sem):
            pltpu.async_copy(
                logits_hbm.at[pl.ds(cid * groups, groups)],
                lbufs[b], sem)
            pltpu.async_copy(
                a_hbm.at[pl.ds(cid * chunk, chunk)],
                abufs[b], sem)

        def wait_copy(cid, b, sem):
            pltpu.make_async_copy(
                logits_hbm.at[pl.ds(cid * groups, groups)],
                lbufs[b], sem).wait()
            pltpu.make_async_copy(
                a_hbm.at[pl.ds(cid * chunk, chunk)],
                abufs[b], sem).wait()

        def compute(b):
            lref = lbufs[b]
            aref = abufs[b]

            @plsc.parallel_loop(0, groups, 1, unroll=4)
            def group_body(i):
                row = jnp.zeros((_L,), jnp.int32) + i
                a = aref[pl.ds(i * _L, _L)]
                es = [jnp.exp(plsc.load_gather(lref, [row, cols[j]]))
                      for j in range(_K)]
                s = ((es[0] + es[1]) + (es[2] + es[3])) + \
                    ((es[4] + es[5]) + (es[6] + es[7]))
                ct = plsc.load_gather(lref, [row, i8 + a])
                err = 1.0 - jnp.exp(ct) / s
                t = i8 + a
                plsc.addupdate_scatter(acc_tbl, [t], err)
                plsc.addupdate_scatter(cnt_tbl, [t], ones)

        start_copy(wid, 0, sem0)

        def pair_body(p, carry):
            c_even = wid + _NW * (2 * p)
            c_odd = c_even + _NW
            start_copy(c_odd, 1, sem1)
            wait_copy(c_even, 0, sem0)
            compute(0)

            @pl.when(p < npairs - 1)
            def _prefetch_even():
                start_copy(c_even + 2 * _NW, 0, sem0)

            wait_copy(c_odd, 1, sem1)
            compute(1)
            return carry

        lax.fori_loop(0, npairs, pair_body, 0)

        if tail:
            @pl.when(wid < tail)
            def _tail_chunk():
                cid = main_chunks + wid
                start_copy(cid, 0, sem0)
                wait_copy(cid, 0, sem0)
                compute(0)

        accv = zeros
        cntv = zeros
        for aa in range(_K):
            ra = jnp.sum(plsc.load_gather(acc_tbl, [i8 + aa]))
            rc = jnp.sum(plsc.load_gather(cnt_tbl, [i8 + aa]))
            accv = jnp.where(iota == aa, ra, accv)
            cntv = jnp.where(iota == _K + aa, rc, cntv)
        obuf[...] = accv + cntv
        pltpu.sync_copy(obuf, out_hbm.at[wid])

    return sc_pass


def _finish_body(p_ref, o_ref):
    x = p_ref[...]
    s = jnp.sum(x, axis=0, keepdims=True)
    acc = s[:, 0:_K]
    cnt = s[:, _K:2 * _K]
    per = jnp.where(cnt > 0.0, acc / jnp.where(cnt > 0.0, cnt, 1.0), acc)
    o_ref[0, 0] = jnp.sum(per) - 1.0


_finish = pl.pallas_call(
    _finish_body,
    out_shape=jax.ShapeDtypeStruct((1, 1), jnp.float32),
    out_specs=pl.BlockSpec(memory_space=pltpu.SMEM),
)


def kernel(adv_logits, A):
    n, k = adv_logits.shape
    wide = adv_logits.reshape(n * k // _W, _W)
    partials = _make_sc_pass(n, 200)(wide, A.astype(jnp.int32))
    return _finish(partials)[0, 0]


# speedup vs baseline: 18.3399x; 18.3399x over previous
import functools

import jax
import jax.numpy as jnp
from jax import lax
from jax.experimental import pallas as pl
from jax.experimental.pallas import tpu as pltpu
from jax.experimental.pallas import tpu_sc as plsc

_NC = 2
_NS = 16
_NW = _NC * _NS
_L = 16
_K = 8
_W = _L * _K


@functools.lru_cache(maxsize=None)
def _make_sc_pass(n_rows: int, groups: int):
    chunk = groups * _L
    total_chunks = n_rows // chunk
    main_chunks = (total_chunks // _NW) * _NW
    per_w = main_chunks // _NW
    npairs = per_w // 2
    tail = total_chunks - main_chunks
    assert total_chunks * chunk == n_rows
    assert npairs * 2 == per_w
    assert tail < _NW

    mesh = plsc.VectorSubcoreMesh(core_axis_name="c", subcore_axis_name="s")

    @functools.partial(
        pl.kernel,
        mesh=mesh,
        out_type=jax.ShapeDtypeStruct((_NW, _L), jnp.float32),
        scratch_types=[
            pltpu.VMEM((_K, chunk), jnp.float32),
            pltpu.VMEM((_K, chunk), jnp.float32),
            pltpu.VMEM((chunk,), jnp.int32),
            pltpu.VMEM((chunk,), jnp.int32),
            pltpu.VMEM((_L * _K,), jnp.float32),
            pltpu.VMEM((_L * _K,), jnp.float32),
            pltpu.VMEM((_L,), jnp.float32),
            pltpu.SemaphoreType.DMA,
            pltpu.SemaphoreType.DMA,
        ],
        compiler_params=pltpu.CompilerParams(needs_layout_passes=False),
    )
    def sc_pass(logits_hbm, a_hbm, out_hbm,
                lbuf0, lbuf1, abuf0, abuf1,
                acc_tbl, cnt_tbl, obuf, sem0, sem1):
        lbufs = (lbuf0, lbuf1)
        abufs = (abuf0, abuf1)
        wid = lax.axis_index("s") * _NC + lax.axis_index("c")

        iota = lax.iota(jnp.int32, _L)
        i8 = iota * _K
        zeros = jnp.zeros((_L,), jnp.float32)
        ones = jnp.ones((_L,), jnp.float32)

        for k in range(_K):
            acc_tbl[pl.ds(k * _L, _L)] = zeros
            cnt_tbl[pl.ds(k * _L, _L)] = zeros

        def start_copy(cid, b, sem):
            pltpu.async_copy(
                logits_hbm.at[:, pl.ds(cid * chunk, chunk)],
                lbufs[b], sem)
            pltpu.async_copy(
                a_hbm.at[pl.ds(cid * chunk, chunk)],
                abufs[b], sem)

        def wait_copy(cid, b, sem):
            pltpu.make_async_copy(
                logits_hbm.at[:, pl.ds(cid * chunk, chunk)],
                lbufs[b], sem).wait()
            pltpu.make_async_copy(
                a_hbm.at[pl.ds(cid * chunk, chunk)],
                abufs[b], sem).wait()

        def compute(b):
            lref = lbufs[b]
            aref = abufs[b]

            @plsc.parallel_loop(0, groups, 1, unroll=4)
            def group_body(i):
                r0 = i * _L
                rows = iota + r0
                a = aref[pl.ds(r0, _L)]
                es = [jnp.exp(lref[j, pl.ds(r0, _L)]) for j in range(_K)]
                s = ((es[0] + es[1]) + (es[2] + es[3])) + \
                    ((es[4] + es[5]) + (es[6] + es[7]))
                ct = plsc.load_gather(lref, [a, rows])
                err = 1.0 - jnp.exp(ct) / s
                t = i8 + a
                plsc.addupdate_scatter(acc_tbl, [t], err)
                plsc.addupdate_scatter(cnt_tbl, [t], ones)

        start_copy(wid, 0, sem0)

        def pair_body(p, carry):
            c_even = wid + _NW * (2 * p)
            c_odd = c_even + _NW
            start_copy(c_odd, 1, sem1)
            wait_copy(c_even, 0, sem0)
            compute(0)

            @pl.when(p < npairs - 1)
            def _prefetch_even():
                start_copy(c_even + 2 * _NW, 0, sem0)

            wait_copy(c_odd, 1, sem1)
            compute(1)
            return carry

        lax.fori_loop(0, npairs, pair_body, 0)

        if tail:
            @pl.when(wid < tail)
            def _tail_chunk():
                cid = main_chunks + wid
                start_copy(cid, 0, sem0)
                wait_copy(cid, 0, sem0)
                compute(0)

        accv = zeros
        cntv = zeros
        for aa in range(_K):
            ra = jnp.sum(plsc.load_gather(acc_tbl, [i8 + aa]))
            rc = jnp.sum(plsc.load_gather(cnt_tbl, [i8 + aa]))
            accv = jnp.where(iota == aa, ra, accv)
            cntv = jnp.where(iota == _K + aa, rc, cntv)
        obuf[...] = accv + cntv
        pltpu.sync_copy(obuf, out_hbm.at[wid])

    return sc_pass


def _finish_body(p_ref, o_ref):
    x = p_ref[...]
    s = jnp.sum(x, axis=0, keepdims=True)
    acc = s[:, 0:_K]
    cnt = s[:, _K:2 * _K]
    per = jnp.where(cnt > 0.0, acc / jnp.where(cnt > 0.0, cnt, 1.0), acc)
    o_ref[0, 0] = jnp.sum(per) - 1.0


_finish = pl.pallas_call(
    _finish_body,
    out_shape=jax.ShapeDtypeStruct((1, 1), jnp.float32),
    out_specs=pl.BlockSpec(memory_space=pltpu.SMEM),
)


def kernel(adv_logits, A):
    n, k = adv_logits.shape
    partials = _make_sc_pass(n, 200)(adv_logits.T, A.astype(jnp.int32))
    return _finish(partials)[0, 0]
